# baseline (speedup 1.0000x reference)
import functools
import jax
import jax.numpy as jnp
from jax import lax
from jax.experimental import pallas as pl
from jax.experimental.pallas import tpu as pltpu
from jax.experimental.pallas import tpu_sc as plsc

G = 512
D = 512
V_TOTAL = 100000
NC = 2
NS = 16
NW = NC * NS
SEG_PER_W = G // NW
CHUNK = 64
NL = D // 16
_NEG = -3.0e38


def _sc_kernel(h_hbm, w_hbm, starts_hbm, out_hbm, rows_v, w_v, starts_v,
               acc_v, out_v):
    wid = lax.axis_index("s") * NC + lax.axis_index("c")
    g0 = wid * SEG_PER_W

    pltpu.sync_copy(w_hbm, w_v)
    pltpu.sync_copy(starts_hbm, starts_v)

    def seg_body(gi, _):
        g = g0 + gi
        sv = starts_v[pl.ds(g, 16)]
        s_row = sv[0]
        e_row = sv[1]

        for c in range(NL):
            acc_v[pl.ds(c * 16, 16)] = jnp.zeros((16,), jnp.float32)

        a0 = (s_row // 8) * 8
        nchunks = (e_row - a0 + CHUNK - 1) // CHUNK

        def chunk_body(k, carry):
            m_run, den = carry
            base = a0 + k * CHUNK
            w0 = jnp.minimum(base, V_TOTAL - CHUNK)
            r_lo = jnp.maximum(s_row, base) - w0
            r_hi = jnp.minimum(e_row, base + CHUNK) - w0
            pltpu.sync_copy(h_hbm.at[pl.ds(w0, CHUNK)], rows_v)

            def row_body(off, rcarry):
                m_prev, den_prev = rcarry
                part = jnp.zeros((16,), jnp.float32)
                for c in range(NL):
                    part = part + rows_v[off, pl.ds(c * 16, 16)] * \
                        w_v[pl.ds(c * 16, 16)]
                s_val = part[0]
                for lane in range(1, 16):
                    s_val = s_val + part[lane]
                m_new = jnp.maximum(m_prev, s_val)
                fvec = jnp.exp(jnp.full((16,), m_prev - m_new, jnp.float32))
                pvec = jnp.exp(jnp.full((16,), s_val - m_new, jnp.float32))
                den_new = den_prev * fvec + pvec

                @pl.when(m_new > m_prev)
                def _rescale():
                    for c in range(NL):
                        acc_v[pl.ds(c * 16, 16)] = \
                            acc_v[pl.ds(c * 16, 16)] * fvec

                for c in range(NL):
                    acc_v[pl.ds(c * 16, 16)] = acc_v[pl.ds(c * 16, 16)] + \
                        pvec * rows_v[off, pl.ds(c * 16, 16)]
                return m_new, den_new

            return lax.fori_loop(r_lo, r_hi, row_body, (m_run, den))

        m_fin, den_fin = lax.fori_loop(
            0, nchunks, chunk_body,
            (jnp.float32(_NEG), jnp.zeros((16,), jnp.float32)))
        del m_fin
        inv = jnp.where(den_fin > 0.0, 1.0 / den_fin, 0.0)
        for c in range(NL):
            out_v[gi, pl.ds(c * 16, 16)] = acc_v[pl.ds(c * 16, 16)] * inv
        return 0

    lax.fori_loop(0, SEG_PER_W, seg_body, 0)
    pltpu.sync_copy(out_v, out_hbm.at[pl.ds(g0, SEG_PER_W)])


_sc_call = functools.partial(
    pl.kernel,
    mesh=plsc.VectorSubcoreMesh(core_axis_name="c", subcore_axis_name="s"),
    out_type=jax.ShapeDtypeStruct((G, D), jnp.float32),
    scratch_types=[
        pltpu.VMEM((CHUNK, D), jnp.float32),
        pltpu.VMEM((D,), jnp.float32),
        pltpu.VMEM((G + 32,), jnp.int32),
        pltpu.VMEM((D,), jnp.float32),
        pltpu.VMEM((SEG_PER_W, D), jnp.float32),
    ],
)(_sc_kernel)


@jax.jit
def kernel(H, batch, W, b):
    del b
    batch = batch.astype(jnp.int32)
    starts = jnp.searchsorted(batch, jnp.arange(G + 32, dtype=jnp.int32))
    starts = jnp.minimum(starts, V_TOTAL).astype(jnp.int32)
    return _sc_call(H, W.reshape(D), starts)


# speedup vs baseline: 11.0487x; 11.0487x over previous
import jax
import jax.numpy as jnp
from jax.experimental import pallas as pl
from jax.experimental.pallas import tpu as pltpu

NUM_SEGMENTS = 512
BV = 5000
LSEG = 64
_CLAMP = 80.0


def _agg_kernel(
    h_ref, batch_ref, w_ref, b_ref, bounds_ref, out_ref, m_ref, den_ref, acc_ref
):
    i = pl.program_id(0)
    nb = pl.num_programs(0)

    h_bf = h_ref[...].astype(jnp.bfloat16)
    batch = batch_ref[0]
    scores = jnp.dot(h_bf, w_ref[...], preferred_element_type=jnp.float32)
    scores_row = scores.reshape(1, BV) + b_ref[0, 0]

    @pl.when(i == 0)
    def _init():
        m_ref[0, 0] = jnp.max(scores_row)
        m_ref[0, 1] = 1.0
        den_ref[...] = jnp.zeros_like(den_ref)
        acc_ref[...] = jnp.zeros_like(acc_ref)

    m_prev = m_ref[0, 0]
    scale = m_ref[0, 1]
    lo8 = bounds_ref[i, 0] * 8
    span = bounds_ref[i, 1]

    p_row = jnp.exp(jnp.minimum(scores_row - m_prev, _CLAMP))

    @pl.when(scale < 1.0)
    def _rescale():
        acc_ref[...] = acc_ref[...] * scale
        den_ref[...] = den_ref[...] * scale

    @pl.when(span < LSEG)
    def _local():
        loc = jax.lax.broadcasted_iota(jnp.int32, (LSEG, BV), 0)
        P = jnp.where(loc == batch - lo8, p_row, 0.0).astype(jnp.bfloat16)
        upd = jnp.dot(P, h_bf, preferred_element_type=jnp.float32)
        ones = jnp.ones((BV, 128), jnp.bfloat16)
        dupd = jnp.dot(P, ones, preferred_element_type=jnp.float32)
        acc_ref[pl.ds(lo8, LSEG), :] += upd
        den_ref[pl.ds(lo8, LSEG), :] += dupd[:, :1]

    @pl.when(span >= LSEG)
    def _full():
        seg_ids = jax.lax.broadcasted_iota(jnp.int32, (NUM_SEGMENTS, BV), 0)
        P = jnp.where(seg_ids == batch, p_row, 0.0).astype(jnp.bfloat16)
        upd = jnp.dot(P, h_bf, preferred_element_type=jnp.float32)
        ones = jnp.ones((BV, 128), jnp.bfloat16)
        dupd = jnp.dot(P, ones, preferred_element_type=jnp.float32)
        acc_ref[...] += upd
        den_ref[...] += dupd[:, :1]

    m_new = jnp.maximum(m_prev, jnp.max(scores_row))
    m_ref[0, 0] = m_new
    m_ref[0, 1] = jnp.exp(m_prev - m_new)

    @pl.when(i == nb - 1)
    def _fini():
        den = den_ref[...]
        out_ref[...] = jnp.where(den > 0.0, acc_ref[...] / den, 0.0)


@jax.jit
def kernel(H, batch, W, b):
    V, D = H.shape
    nb = (V + BV - 1) // BV
    vpad = nb * BV - V
    batch = batch.astype(jnp.int32)
    if vpad:
        H = jnp.concatenate([H, jnp.zeros((vpad, D), H.dtype)], axis=0)
        batch = jnp.concatenate(
            [batch, jnp.full((vpad,), NUM_SEGMENTS, jnp.int32)]
        )
    batch_r = batch.reshape(nb, 1, BV)
    b_r = b.reshape(1, 1).astype(jnp.float32)
    w_bf = W.astype(jnp.bfloat16)

    lo8 = jnp.minimum((batch_r[:, 0, 0] // 8) * 8, NUM_SEGMENTS - LSEG)
    span = batch_r[:, 0, -1] - lo8
    bounds = jnp.stack([lo8 // 8, span], axis=1)

    out = pl.pallas_call(
        _agg_kernel,
        grid=(nb,),
        in_specs=[
            pl.BlockSpec((BV, D), lambda i: (i, 0)),
            pl.BlockSpec((1, 1, BV), lambda i: (i, 0, 0)),
            pl.BlockSpec((D, 1), lambda i: (0, 0)),
            pl.BlockSpec((1, 1), lambda i: (0, 0)),
            pl.BlockSpec((nb, 2), lambda i: (0, 0), memory_space=pltpu.SMEM),
        ],
        out_specs=pl.BlockSpec((NUM_SEGMENTS, D), lambda i: (0, 0)),
        out_shape=jax.ShapeDtypeStruct((NUM_SEGMENTS, D), jnp.float32),
        scratch_shapes=[
            pltpu.SMEM((1, 2), jnp.float32),
            pltpu.VMEM((NUM_SEGMENTS, 1), jnp.float32),
            pltpu.VMEM((NUM_SEGMENTS, D), jnp.float32),
        ],
    )(H, batch_r, w_bf, b_r, bounds)
    return out


# speedup vs baseline: 11.1058x; 1.0052x over previous
import jax
import jax.numpy as jnp
from jax.experimental import pallas as pl
from jax.experimental.pallas import tpu as pltpu

NUM_SEGMENTS = 512
BV = 5000
LSEG = 128
_CLAMP = 80.0


def _agg_kernel(
    h_ref, batch_ref, w_ref, b_ref, bounds_ref, out_ref, m_ref, den_ref, acc_ref
):
    i = pl.program_id(0)
    nb = pl.num_programs(0)

    h_bf = h_ref[...].astype(jnp.bfloat16)
    batch = batch_ref[0]
    scores = jnp.dot(h_bf, w_ref[...], preferred_element_type=jnp.float32)
    scores_row = scores.reshape(1, BV) + b_ref[0, 0]

    @pl.when(i == 0)
    def _init():
        m_ref[0, 0] = jnp.max(scores_row)
        m_ref[0, 1] = 1.0
        den_ref[...] = jnp.zeros_like(den_ref)
        acc_ref[...] = jnp.zeros_like(acc_ref)

    m_prev = m_ref[0, 0]
    scale = m_ref[0, 1]
    lo8 = bounds_ref[i, 0] * 8
    span = bounds_ref[i, 1]

    p_row = jnp.exp(jnp.minimum(scores_row - m_prev, _CLAMP))

    @pl.when(scale < 1.0)
    def _rescale():
        acc_ref[...] = acc_ref[...] * scale
        den_ref[...] = den_ref[...] * scale

    @pl.when(span < LSEG)
    def _local():
        loc = jax.lax.broadcasted_iota(jnp.int32, (LSEG, BV), 0)
        P = jnp.where(loc == batch - lo8, p_row, 0.0).astype(jnp.bfloat16)
        upd = jnp.dot(P, h_bf, preferred_element_type=jnp.float32)
        ones = jnp.ones((BV, 128), jnp.bfloat16)
        dupd = jnp.dot(P, ones, preferred_element_type=jnp.float32)
        acc_ref[pl.ds(lo8, LSEG), :] += upd
        den_ref[pl.ds(lo8, LSEG), :] += dupd[:, :1]

    @pl.when(span >= LSEG)
    def _full():
        seg_ids = jax.lax.broadcasted_iota(jnp.int32, (NUM_SEGMENTS, BV), 0)
        P = jnp.where(seg_ids == batch, p_row, 0.0).astype(jnp.bfloat16)
        upd = jnp.dot(P, h_bf, preferred_element_type=jnp.float32)
        ones = jnp.ones((BV, 128), jnp.bfloat16)
        dupd = jnp.dot(P, ones, preferred_element_type=jnp.float32)
        acc_ref[...] += upd
        den_ref[...] += dupd[:, :1]

    m_new = jnp.maximum(m_prev, jnp.max(scores_row))
    m_ref[0, 0] = m_new
    m_ref[0, 1] = jnp.exp(m_prev - m_new)

    @pl.when(i == nb - 1)
    def _fini():
        den = den_ref[...]
        out_ref[...] = jnp.where(den > 0.0, acc_ref[...] / den, 0.0)


@jax.jit
def kernel(H, batch, W, b):
    V, D = H.shape
    nb = (V + BV - 1) // BV
    vpad = nb * BV - V
    batch = batch.astype(jnp.int32)
    if vpad:
        H = jnp.concatenate([H, jnp.zeros((vpad, D), H.dtype)], axis=0)
        batch = jnp.concatenate(
            [batch, jnp.full((vpad,), NUM_SEGMENTS, jnp.int32)]
        )
    batch_r = batch.reshape(nb, 1, BV)
    b_r = b.reshape(1, 1).astype(jnp.float32)
    w_bf = W.astype(jnp.bfloat16)

    lo8 = jnp.minimum((batch_r[:, 0, 0] // 8) * 8, NUM_SEGMENTS - LSEG)
    span = batch_r[:, 0, -1] - lo8
    bounds = jnp.stack([lo8 // 8, span], axis=1)

    out = pl.pallas_call(
        _agg_kernel,
        grid=(nb,),
        in_specs=[
            pl.BlockSpec((BV, D), lambda i: (i, 0)),
            pl.BlockSpec((1, 1, BV), lambda i: (i, 0, 0)),
            pl.BlockSpec((D, 1), lambda i: (0, 0)),
            pl.BlockSpec((1, 1), lambda i: (0, 0)),
            pl.BlockSpec((nb, 2), lambda i: (0, 0), memory_space=pltpu.SMEM),
        ],
        out_specs=pl.BlockSpec((NUM_SEGMENTS, D), lambda i: (0, 0)),
        out_shape=jax.ShapeDtypeStruct((NUM_SEGMENTS, D), jnp.float32),
        scratch_shapes=[
            pltpu.SMEM((1, 2), jnp.float32),
            pltpu.VMEM((NUM_SEGMENTS, 1), jnp.float32),
            pltpu.VMEM((NUM_SEGMENTS, D), jnp.float32),
        ],
    )(H, batch_r, w_bf, b_r, bounds)
    return out
